# baseline (speedup 1.0000x reference)
import functools

import jax
import jax.numpy as jnp
from jax import lax
from jax.experimental import pallas as pl
from jax.experimental.pallas import tpu as pltpu
from jax.experimental.pallas import tpu_sc as plsc

EMBED_DIM = 64
IDXW = 128
CHUNK = 1024
NGRP = CHUNK // IDXW


def _make_lookup(B, D):
    info = plsc.get_sparse_core_info()
    NC, NS = info.num_cores, info.num_subcores
    NW = NC * NS
    assert B % (NW * CHUNK) == 0
    b_per_w = B // NW
    n_chunks = b_per_w // CHUNK
    mesh = plsc.VectorSubcoreMesh(core_axis_name="c", subcore_axis_name="s")

    @functools.partial(
        pl.kernel,
        mesh=mesh,
        out_type=jax.ShapeDtypeStruct((B, D), jnp.float32),
        scratch_types=[
            pltpu.VMEM((NGRP, IDXW), jnp.int32),
            pltpu.VMEM((CHUNK, D), jnp.float32),
            pltpu.SemaphoreType.DMA,
        ],
        compiler_params=pltpu.CompilerParams(use_tc_tiling_on_sc=False),
    )
    def lookup(x_hbm, table_hbm, out_hbm, idx_v, rows_v, sem):
        wid = lax.axis_index("s") * NC + lax.axis_index("c")
        tile_base = wid * b_per_w

        def chunk_body(g, carry):
            row0 = pl.multiple_of(tile_base + g * CHUNK, CHUNK)
            pltpu.sync_copy(
                x_hbm.at[pl.ds(pl.multiple_of(row0 // IDXW, NGRP), NGRP)], idx_v
            )
            descs = [
                pltpu.async_copy(
                    table_hbm.at[idx_v.at[j]],
                    rows_v.at[pl.ds(j * IDXW, IDXW)],
                    sem,
                )
                for j in range(NGRP)
            ]
            for d in descs:
                d.wait()
            pltpu.sync_copy(rows_v, out_hbm.at[pl.ds(row0, CHUNK)])
            return carry

        lax.fori_loop(0, n_chunks, chunk_body, 0)

    return lookup


def kernel(x, table):
    orig_shape = x.shape
    B = x.size
    D = table.shape[1]
    x_flat = x.reshape(B // IDXW, IDXW).astype(jnp.int32)
    out = _make_lookup(B, D)(x_flat, table)
    return out.reshape(*orig_shape, D)


# speedup vs baseline: 1.0205x; 1.0205x over previous
import functools

import jax
import jax.numpy as jnp
from jax import lax
from jax.experimental import pallas as pl
from jax.experimental.pallas import tpu as pltpu
from jax.experimental.pallas import tpu_sc as plsc

IDXW = 128
SUB = 512
NGRP = SUB // IDXW


def _make_lookup(B, D):
    info = plsc.get_sparse_core_info()
    NC, NS = info.num_cores, info.num_subcores
    NW = NC * NS
    assert B % (NW * 2 * SUB) == 0
    b_per_w = B // NW
    irows = b_per_w // IDXW
    n_sub = b_per_w // SUB
    mesh = plsc.VectorSubcoreMesh(core_axis_name="c", subcore_axis_name="s")

    @functools.partial(
        pl.kernel,
        mesh=mesh,
        out_type=jax.ShapeDtypeStruct((B, D), jnp.float32),
        scratch_types=[
            pltpu.VMEM((irows, IDXW), jnp.int32),
            pltpu.VMEM((SUB, D), jnp.float32),
            pltpu.VMEM((SUB, D), jnp.float32),
            pltpu.SemaphoreType.DMA,
            pltpu.SemaphoreType.DMA,
        ],
        compiler_params=pltpu.CompilerParams(use_tc_tiling_on_sc=False),
    )
    def lookup(x_hbm, table_hbm, out_hbm, idx_all, rows0, rows1, sg0, sg1):
        wid = lax.axis_index("s") * NC + lax.axis_index("c")
        tile_base = wid * b_per_w
        pltpu.sync_copy(
            x_hbm.at[pl.ds(pl.multiple_of(wid * irows, 8), irows)], idx_all
        )

        rows = (rows0, rows1)
        sg = (sg0, sg1)

        def fire_gathers(t, b):
            for j in range(NGRP):
                pltpu.async_copy(
                    table_hbm.at[idx_all.at[t * NGRP + j]],
                    rows[b].at[pl.ds(j * IDXW, IDXW)],
                    sg[b],
                )

        def wait_gathers(b):
            pltpu.make_async_copy(out_hbm.at[pl.ds(0, SUB)], rows[b], sg[b]).wait()

        def write_back(t, b):
            row0 = pl.multiple_of(tile_base + t * SUB, SUB)
            pltpu.sync_copy(rows[b], out_hbm.at[pl.ds(row0, SUB)])

        fire_gathers(0, 0)
        fire_gathers(1, 1)

        def pair(k, carry):
            for b in range(2):
                t = 2 * k + b
                wait_gathers(b)
                write_back(t, b)
                fire_gathers(t + 2, b)
            return carry

        lax.fori_loop(0, n_sub // 2 - 1, pair, 0)
        for b in range(2):
            t = n_sub - 2 + b
            wait_gathers(b)
            write_back(t, b)

    return lookup


def kernel(x, table):
    orig_shape = x.shape
    B = x.size
    D = table.shape[1]
    x_flat = x.reshape(B // IDXW, IDXW).astype(jnp.int32)
    out = _make_lookup(B, D)(x_flat, table)
    return out.reshape(*orig_shape, D)
